# baseline (speedup 1.0000x reference)
import functools

import jax
import jax.numpy as jnp
from jax import lax
from jax.experimental import pallas as pl
from jax.experimental.pallas import tpu as pltpu
from jax.experimental.pallas import tpu_sc as plsc

NC = 2
NS = 16
L = 16


def _sc_scores(center2d, context2d, neg2d, in_w, out_w, B, K, D):
    NW = NC * NS
    BPW = B // NW
    NEG_PW = BPW * K
    IDXW = 128
    CROWS = B // IDXW // NW
    NROWS = NEG_PW // IDXW
    BLK_B = 32
    RPB = BLK_B * K
    DPB = RPB // IDXW
    NBLK = BPW // BLK_B

    mesh = plsc.VectorSubcoreMesh(
        core_axis_name="c", subcore_axis_name="s",
        num_cores=NC, num_subcores=NS)

    @functools.partial(
        pl.kernel,
        out_type=(jax.ShapeDtypeStruct((B,), jnp.float32),
                  jax.ShapeDtypeStruct((B * K,), jnp.float32)),
        mesh=mesh,
        compiler_params=pltpu.CompilerParams(
            needs_layout_passes=False, use_tc_tiling_on_sc=False),
        scratch_types=[
            pltpu.VMEM((CROWS, IDXW), jnp.int32),
            pltpu.VMEM((CROWS, IDXW), jnp.int32),
            pltpu.VMEM((NROWS, IDXW), jnp.int32),
            pltpu.VMEM((BPW, D), jnp.float32),
            pltpu.VMEM((BPW, D), jnp.float32),
            pltpu.VMEM((2, RPB, D), jnp.float32),
            pltpu.VMEM((BPW,), jnp.float32),
            pltpu.VMEM((NEG_PW,), jnp.float32),
            pltpu.SemaphoreType.DMA,
            pltpu.SemaphoreType.DMA,
            pltpu.SemaphoreType.DMA,
        ],
    )
    def sc_kernel(in_w_h, out_w_h, c2_h, x2_h, n2_h, pos_h, neg_h,
                  cidx, xidx, nidx, vrows, prows, nbuf, posv, negv,
                  sem_vp, semn0, semn1):
        wid = lax.axis_index("s") * NC + lax.axis_index("c")

        pltpu.sync_copy(c2_h.at[pl.ds(wid * CROWS, CROWS)], cidx)
        pltpu.sync_copy(x2_h.at[pl.ds(wid * CROWS, CROWS)], xidx)
        pltpu.sync_copy(n2_h.at[pl.ds(wid * NROWS, NROWS)], nidx)

        vp_copies = []
        for j in range(CROWS):
            vp_copies.append(pltpu.async_copy(
                in_w_h.at[cidx.at[j]], vrows.at[pl.ds(j * IDXW, IDXW)], sem_vp))
            vp_copies.append(pltpu.async_copy(
                out_w_h.at[xidx.at[j]], prows.at[pl.ds(j * IDXW, IDXW)], sem_vp))

        def issue_neg(g, slot, sem):
            for j in range(DPB):
                pltpu.async_copy(
                    out_w_h.at[nidx.at[g * DPB + j]],
                    nbuf.at[slot, pl.ds(j * IDXW, IDXW)], sem)

        issue_neg(0, 0, semn0)
        issue_neg(1, 1, semn1)
        for c in vp_copies:
            c.wait()

        i16 = lax.iota(jnp.int32, L)
        i20 = i16 * K

        @pl.loop(0, NBLK, step=2)
        def _blk(go):
            for par in range(2):
                g = go + par
                nsem = semn0 if par == 0 else semn1
                pltpu.make_async_copy(
                    out_w_h.at[pl.ds(0, RPB)], nbuf.at[par], nsem).wait()
                for sb in range(2):
                    b0 = g * BLK_B + sb * L
                    bvec = b0 + i16
                    vcols = [plsc.load_gather(
                        vrows, [bvec, jnp.full((L,), d, jnp.int32)])
                        for d in range(D)]
                    pa = [jnp.zeros((L,), jnp.float32) for _ in range(4)]
                    for d in range(D):
                        u = plsc.load_gather(
                            prows, [bvec, jnp.full((L,), d, jnp.int32)])
                        pa[d % 4] = pa[d % 4] + u * vcols[d]
                    posv[pl.ds(b0, L)] = (pa[0] + pa[1]) + (pa[2] + pa[3])
                    rbase = sb * L * K
                    obase = b0 * K

                    @pl.loop(0, K)
                    def _k(k):
                        rv = rbase + i20 + k
                        na = [jnp.zeros((L,), jnp.float32) for _ in range(4)]
                        for d in range(D):
                            u = plsc.load_gather(
                                nbuf.at[par],
                                [rv, jnp.full((L,), d, jnp.int32)])
                            na[d % 4] = na[d % 4] + u * vcols[d]
                        plsc.store_scatter(
                            negv, [obase + i20 + k],
                            (na[0] + na[1]) + (na[2] + na[3]))

                @pl.when(g + 2 < NBLK)
                def _():
                    issue_neg(g + 2, par, nsem)

        pltpu.sync_copy(posv, pos_h.at[pl.ds(wid * BPW, BPW)])
        pltpu.sync_copy(negv, neg_h.at[pl.ds(wid * NEG_PW, NEG_PW)])

    return sc_kernel(in_w, out_w, center2d, context2d, neg2d)


def _loss_reduce(pos_score, neg_flat, B):
    def body(pos_ref, neg_ref, out_ref):
        def logsig(x):
            return jnp.minimum(x, 0.0) - jnp.log1p(jnp.exp(-jnp.abs(x)))
        s = jnp.sum(logsig(pos_ref[...])) + jnp.sum(logsig(-neg_ref[...]))
        out_ref[0, 0] = -s / B

    out = pl.pallas_call(
        body,
        out_shape=jax.ShapeDtypeStruct((1, 1), jnp.float32),
        out_specs=pl.BlockSpec(memory_space=pltpu.SMEM),
    )(pos_score.reshape(B // 128, 128),
      neg_flat.reshape(-1, 128))
    return out.reshape(())


def kernel(center, context, negative, in_embed_w, out_embed_w):
    B, = center.shape
    K = negative.shape[1]
    D = in_embed_w.shape[1]
    center2d = center.astype(jnp.int32).reshape(B // 128, 128)
    context2d = context.astype(jnp.int32).reshape(B // 128, 128)
    neg2d = negative.astype(jnp.int32).reshape(B * K // 128, 128)
    pos_score, neg_flat = _sc_scores(
        center2d, context2d, neg2d, in_embed_w, out_embed_w, B, K, D)
    return _loss_reduce(pos_score, neg_flat, B)
